# Initial kernel scaffold; baseline (speedup 1.0000x reference)
import jax
import jax.numpy as jnp
from jax.experimental import pallas as pl


def kernel(x, alpha, delta, root, ema_weights):
    raise NotImplementedError("write your pallas kernel here")


# speedup vs baseline: 26.6142x; 26.6142x over previous
import functools

import jax
import jax.numpy as jnp
from jax.experimental import pallas as pl
from jax.experimental.pallas import tpu as pltpu

_EPS = 1e-06
_LANE = 128


def _pcen_body(params_ref, u_ref, x_ref, o_ref, carry_ref, p_ref, winvp_ref):
    k = pl.program_id(1)
    x = x_ref[...]

    @pl.when(k == 0)
    def _init():
        w = params_ref[:, 0:1]
        a = params_ref[:, 1:2]
        la = jnp.log(a)
        i = jax.lax.broadcasted_iota(jnp.int32, (1, _LANE), 1).astype(jnp.float32)
        p = jnp.exp(i * la)
        p_ref[...] = p
        winvp_ref[...] = w / p
        carry_ref[...] = x[:, 0:1]

    a = params_ref[:, 1:2]
    alpha = params_ref[:, 2:3]
    inv_r = params_ref[:, 3:4]
    delta = params_ref[:, 4:5]
    dr = params_ref[:, 5:6]

    s = x * winvp_ref[...]
    cum = jax.lax.dot_general(
        s, u_ref[...], (((1,), (0,)), ((), ())),
        preferred_element_type=jnp.float32)
    ema = p_ref[...] * (a * carry_ref[...] + cum)
    carry_ref[...] = ema[:, _LANE - 1:_LANE]

    den = jnp.exp(-alpha * jnp.log(ema + _EPS))
    q = x * den + delta
    o_ref[...] = jnp.exp(inv_r * jnp.log(q)) - dr


@jax.jit
def kernel(x, alpha, delta, root, ema_weights):
    B, _, C, T = x.shape
    R = B * C
    xr = x.reshape(R, T)

    w = jnp.clip(ema_weights, 0.0, 0.2)
    a = 1.0 - w
    al = jnp.minimum(alpha, 1.0)
    r = jnp.maximum(root, 1.0)
    inv_r = 1.0 / r
    dr = delta ** inv_r
    pc = jnp.stack([w, a, al, inv_r, delta, dr,
                    jnp.zeros_like(w), jnp.zeros_like(w)], axis=1)
    params = jnp.tile(pc, (B, 1))

    u = jnp.triu(jnp.ones((_LANE, _LANE), jnp.float32))

    Rb = 512
    G = R // Rb
    K = T // _LANE

    out = pl.pallas_call(
        _pcen_body,
        grid=(G, K),
        in_specs=[
            pl.BlockSpec((Rb, 8), lambda g, k: (g, 0)),
            pl.BlockSpec((_LANE, _LANE), lambda g, k: (0, 0)),
            pl.BlockSpec((Rb, _LANE), lambda g, k: (g, k)),
        ],
        out_specs=pl.BlockSpec((Rb, _LANE), lambda g, k: (g, k)),
        out_shape=jax.ShapeDtypeStruct((R, T), jnp.float32),
        scratch_shapes=[
            pltpu.VMEM((Rb, 1), jnp.float32),
            pltpu.VMEM((Rb, _LANE), jnp.float32),
            pltpu.VMEM((Rb, _LANE), jnp.float32),
        ],
        compiler_params=pltpu.CompilerParams(
            dimension_semantics=("parallel", "arbitrary"),
        ),
    )(params, u, xr)

    return out.reshape(B, C, T)


# speedup vs baseline: 63.8826x; 2.4003x over previous
import jax
import jax.numpy as jnp
from jax.experimental import pallas as pl
from jax.experimental.pallas import tpu as pltpu

_EPS = 1e-06
_LANE = 128
_NSUB = 4
_RB = 512


def _pcen_body(params_ref, u_ref, x_ref, o_ref,
               acb_ref, p_ref, winvp_ref, na_ref, invr_ref, delta_ref, dr_ref):
    k = pl.program_id(1)
    ones = jnp.zeros((_RB, _LANE), jnp.float32)

    @pl.when(k == 0)
    def _init():
        w = params_ref[:, 0:1]
        a = params_ref[:, 1:2]
        la = jnp.log(a)
        i = jax.lax.broadcasted_iota(jnp.int32, (1, _LANE), 1).astype(jnp.float32)
        p = jnp.exp(i * la)
        p_ref[...] = p
        winvp_ref[...] = w / p
        na_ref[...] = -params_ref[:, 2:3] + ones
        invr_ref[...] = params_ref[:, 3:4] + ones
        delta_ref[...] = params_ref[:, 4:5] + ones
        dr_ref[...] = params_ref[:, 5:6] + ones
        acb_ref[...] = a * x_ref[:, 0:1] + ones

    p = p_ref[...]
    winvp = winvp_ref[...]
    na = na_ref[...]
    invr = invr_ref[...]
    delta = delta_ref[...]
    dr = dr_ref[...]
    a_col = params_ref[:, 1:2]
    u = u_ref[...]

    ac = acb_ref[...]
    for c in range(_NSUB):
        xs = x_ref[:, c * _LANE:(c + 1) * _LANE]
        s = xs * winvp
        cum = jax.lax.dot_general(
            s, u, (((1,), (0,)), ((), ())),
            preferred_element_type=jnp.float32)
        ema = p * (cum + ac)
        ac = jnp.broadcast_to(a_col * ema[:, _LANE - 1:_LANE], (_RB, _LANE))
        den = jnp.exp(na * jnp.log(ema + _EPS))
        q = xs * den + delta
        o_ref[:, c * _LANE:(c + 1) * _LANE] = jnp.exp(invr * jnp.log(q)) - dr
    acb_ref[...] = ac


@jax.jit
def kernel(x, alpha, delta, root, ema_weights):
    B, _, C, T = x.shape
    R = B * C
    xr = x.reshape(R, T)

    w = jnp.clip(ema_weights, 0.0, 0.2)
    a = 1.0 - w
    al = jnp.minimum(alpha, 1.0)
    r = jnp.maximum(root, 1.0)
    inv_r = 1.0 / r
    dr = delta ** inv_r
    pc = jnp.stack([w, a, al, inv_r, delta, dr,
                    jnp.zeros_like(w), jnp.zeros_like(w)], axis=1)
    params = jnp.tile(pc, (B, 1))

    u = jnp.triu(jnp.ones((_LANE, _LANE), jnp.float32))

    TB = _NSUB * _LANE
    G = R // _RB
    K = T // TB

    out = pl.pallas_call(
        _pcen_body,
        grid=(G, K),
        in_specs=[
            pl.BlockSpec((_RB, 8), lambda g, k: (g, 0)),
            pl.BlockSpec((_LANE, _LANE), lambda g, k: (0, 0)),
            pl.BlockSpec((_RB, TB), lambda g, k: (g, k)),
        ],
        out_specs=pl.BlockSpec((_RB, TB), lambda g, k: (g, k)),
        out_shape=jax.ShapeDtypeStruct((R, T), jnp.float32),
        scratch_shapes=[pltpu.VMEM((_RB, _LANE), jnp.float32)] * 7,
        compiler_params=pltpu.CompilerParams(
            dimension_semantics=("parallel", "arbitrary"),
        ),
    )(params, u, xr)

    return out.reshape(B, C, T)


# speedup vs baseline: 67.0014x; 1.0488x over previous
import jax
import jax.numpy as jnp
from jax.experimental import pallas as pl
from jax.experimental.pallas import tpu as pltpu

_EPS = 1e-06
_LANE = 128
_NSUB = 4
_RB = 512


def _pcen_body(params_ref, u_ref, x_ref, o_ref,
               acb_ref, p_ref, winvp_ref, ap_ref,
               na_ref, invr_ref, delta_ref, dr_ref):
    k = pl.program_id(1)
    ones = jnp.zeros((_RB, _LANE), jnp.float32)

    @pl.when(k == 0)
    def _init():
        w = params_ref[:, 0:1]
        a = params_ref[:, 1:2]
        la = jnp.log(a)
        i = jax.lax.broadcasted_iota(jnp.int32, (1, _LANE), 1).astype(jnp.float32)
        p = jnp.exp(i * la)
        p_ref[...] = p
        winvp_ref[...] = w / p
        ap_ref[...] = jnp.exp(128.0 * la) + ones
        na_ref[...] = -params_ref[:, 2:3] + ones
        invr_ref[...] = params_ref[:, 3:4] + ones
        delta_ref[...] = params_ref[:, 4:5] + ones
        dr_ref[...] = params_ref[:, 5:6] + ones
        acb_ref[...] = a * x_ref[:, 0:1] + ones

    p = p_ref[...]
    winvp = winvp_ref[...]
    ap = ap_ref[...]
    na = na_ref[...]
    invr = invr_ref[...]
    delta = delta_ref[...]
    dr = dr_ref[...]
    u = u_ref[...]

    ac = acb_ref[...]
    for c in range(_NSUB):
        xs = x_ref[:, c * _LANE:(c + 1) * _LANE]
        s = xs * winvp
        both = jax.lax.dot_general(
            s, u, (((1,), (0,)), ((), ())),
            preferred_element_type=jnp.float32)
        cum = both[:, :_LANE]
        sb = both[:, _LANE:]
        ema = p * (cum + ac)
        den = jnp.exp(na * jnp.log(ema + _EPS))
        ac = ap * (sb + ac)
        q = xs * den + delta
        o_ref[:, c * _LANE:(c + 1) * _LANE] = jnp.exp(invr * jnp.log(q)) - dr
    acb_ref[...] = ac


@jax.jit
def kernel(x, alpha, delta, root, ema_weights):
    B, _, C, T = x.shape
    R = B * C
    xr = x.reshape(R, T)

    w = jnp.clip(ema_weights, 0.0, 0.2)
    a = 1.0 - w
    al = jnp.minimum(alpha, 1.0)
    r = jnp.maximum(root, 1.0)
    inv_r = 1.0 / r
    dr = delta ** inv_r
    pc = jnp.stack([w, a, al, inv_r, delta, dr,
                    jnp.zeros_like(w), jnp.zeros_like(w)], axis=1)
    params = jnp.tile(pc, (B, 1))

    u = jnp.concatenate([jnp.triu(jnp.ones((_LANE, _LANE), jnp.float32)),
                         jnp.ones((_LANE, _LANE), jnp.float32)], axis=1)

    TB = _NSUB * _LANE
    G = R // _RB
    K = T // TB

    out = pl.pallas_call(
        _pcen_body,
        grid=(G, K),
        in_specs=[
            pl.BlockSpec((_RB, 8), lambda g, k: (g, 0)),
            pl.BlockSpec((_LANE, 2 * _LANE), lambda g, k: (0, 0)),
            pl.BlockSpec((_RB, TB), lambda g, k: (g, k)),
        ],
        out_specs=pl.BlockSpec((_RB, TB), lambda g, k: (g, k)),
        out_shape=jax.ShapeDtypeStruct((R, T), jnp.float32),
        scratch_shapes=[pltpu.VMEM((_RB, _LANE), jnp.float32)] * 8,
        compiler_params=pltpu.CompilerParams(
            dimension_semantics=("parallel", "arbitrary"),
        ),
    )(params, u, xr)

    return out.reshape(B, C, T)


# speedup vs baseline: 86.3207x; 1.2883x over previous
import jax
import jax.numpy as jnp
from jax.experimental import pallas as pl
from jax.experimental.pallas import tpu as pltpu

_EPS = 1e-06
_LANE = 128
_NSUB = 8
_RB = 512


def _pcen_body(params_ref, u_ref, x_ref, o_ref,
               acb_ref, p_ref, winvp_ref, ap_ref,
               na_ref, invr_ref, delta_ref, dr_ref):
    k = pl.program_id(1)
    ones = jnp.zeros((_RB, _LANE), jnp.float32)

    @pl.when(k == 0)
    def _init():
        w = params_ref[:, 0:1]
        a = params_ref[:, 1:2]
        la = jnp.log(a)
        i = jax.lax.broadcasted_iota(jnp.int32, (1, _LANE), 1).astype(jnp.float32)
        p = jnp.exp(i * la)
        p_ref[...] = p
        winvp_ref[...] = w / p
        ap_ref[...] = jnp.exp(128.0 * la) + ones
        na_ref[...] = -params_ref[:, 2:3] + ones
        invr_ref[...] = params_ref[:, 3:4] + ones
        delta_ref[...] = params_ref[:, 4:5] + ones
        dr_ref[...] = params_ref[:, 5:6] + ones
        acb_ref[...] = a * x_ref[:, 0:1] + ones

    p = p_ref[...]
    winvp = winvp_ref[...]
    ap = ap_ref[...]
    na = na_ref[...]
    invr = invr_ref[...]
    delta = delta_ref[...]
    dr = dr_ref[...]
    u = u_ref[...]

    ac = acb_ref[...]
    for c in range(_NSUB):
        xs = x_ref[:, c * _LANE:(c + 1) * _LANE]
        s = xs * winvp
        both = jax.lax.dot_general(
            s, u, (((1,), (0,)), ((), ())),
            preferred_element_type=jnp.float32)
        cum = both[:, :_LANE]
        sb = both[:, _LANE:]
        ema = p * (cum + ac)
        den = jnp.exp2(na * jnp.log2(ema + _EPS))
        ac = ap * (sb + ac)
        q = xs * den + delta
        o_ref[:, c * _LANE:(c + 1) * _LANE] = jnp.exp2(invr * jnp.log2(q)) - dr
    acb_ref[...] = ac


@jax.jit
def kernel(x, alpha, delta, root, ema_weights):
    B, _, C, T = x.shape
    R = B * C
    xr = x.reshape(R, T)

    w = jnp.clip(ema_weights, 0.0, 0.2)
    a = 1.0 - w
    al = jnp.minimum(alpha, 1.0)
    r = jnp.maximum(root, 1.0)
    inv_r = 1.0 / r
    dr = delta ** inv_r
    pc = jnp.stack([w, a, al, inv_r, delta, dr,
                    jnp.zeros_like(w), jnp.zeros_like(w)], axis=1)
    params = jnp.tile(pc, (B, 1))

    u = jnp.concatenate([jnp.triu(jnp.ones((_LANE, _LANE), jnp.float32)),
                         jnp.ones((_LANE, _LANE), jnp.float32)], axis=1)

    TB = _NSUB * _LANE
    G = R // _RB
    K = T // TB

    out = pl.pallas_call(
        _pcen_body,
        grid=(G, K),
        in_specs=[
            pl.BlockSpec((_RB, 8), lambda g, k: (g, 0)),
            pl.BlockSpec((_LANE, 2 * _LANE), lambda g, k: (0, 0)),
            pl.BlockSpec((_RB, TB), lambda g, k: (g, k)),
        ],
        out_specs=pl.BlockSpec((_RB, TB), lambda g, k: (g, k)),
        out_shape=jax.ShapeDtypeStruct((R, T), jnp.float32),
        scratch_shapes=[pltpu.VMEM((_RB, _LANE), jnp.float32)] * 8,
        compiler_params=pltpu.CompilerParams(
            dimension_semantics=("parallel", "arbitrary"),
        ),
    )(params, u, xr)

    return out.reshape(B, C, T)


# speedup vs baseline: 100.7346x; 1.1670x over previous
import jax
import jax.numpy as jnp
from jax.experimental import pallas as pl
from jax.experimental.pallas import tpu as pltpu

_EPS = 1e-06
_LANE = 128
_NSUB = 8
_RB = 1024


def _pcen_body(params_ref, u_ref, x_ref, o_ref,
               acb_ref, p_ref, winvp_ref, ap_ref,
               na_ref, invr_ref, delta_ref, dr_ref):
    k = pl.program_id(1)
    ones = jnp.zeros((_RB, _LANE), jnp.float32)

    @pl.when(k == 0)
    def _init():
        w = params_ref[:, 0:1]
        a = params_ref[:, 1:2]
        la = jnp.log(a)
        i = jax.lax.broadcasted_iota(jnp.int32, (1, _LANE), 1).astype(jnp.float32)
        p = jnp.exp(i * la)
        p_ref[...] = p
        winvp_ref[...] = w / p
        ap_ref[...] = jnp.exp(128.0 * la) + ones
        na_ref[...] = -params_ref[:, 2:3] + ones
        invr_ref[...] = params_ref[:, 3:4] + ones
        delta_ref[...] = params_ref[:, 4:5] + ones
        dr_ref[...] = params_ref[:, 5:6] + ones
        acb_ref[...] = a * x_ref[:, 0:1] + ones

    p = p_ref[...]
    winvp = winvp_ref[...]
    ap = ap_ref[...]
    na = na_ref[...]
    invr = invr_ref[...]
    delta = delta_ref[...]
    dr = dr_ref[...]
    u = u_ref[...]

    ac = acb_ref[...]
    for c in range(_NSUB):
        xs = x_ref[:, c * _LANE:(c + 1) * _LANE]
        s = xs * winvp
        both = jax.lax.dot_general(
            s, u, (((1,), (0,)), ((), ())),
            preferred_element_type=jnp.float32)
        cum = both[:, :_LANE]
        sb = both[:, _LANE:]
        ema = p * (cum + ac)
        den = jnp.exp2(na * jnp.log2(ema + _EPS))
        ac = ap * (sb + ac)
        q = xs * den + delta
        o_ref[:, c * _LANE:(c + 1) * _LANE] = jnp.exp2(invr * jnp.log2(q)) - dr
    acb_ref[...] = ac


@jax.jit
def kernel(x, alpha, delta, root, ema_weights):
    B, _, C, T = x.shape
    R = B * C
    xr = x.reshape(R, T)

    w = jnp.clip(ema_weights, 0.0, 0.2)
    a = 1.0 - w
    al = jnp.minimum(alpha, 1.0)
    r = jnp.maximum(root, 1.0)
    inv_r = 1.0 / r
    dr = delta ** inv_r
    pc = jnp.stack([w, a, al, inv_r, delta, dr,
                    jnp.zeros_like(w), jnp.zeros_like(w)], axis=1)
    params = jnp.tile(pc, (B, 1))

    u = jnp.concatenate([jnp.triu(jnp.ones((_LANE, _LANE), jnp.float32)),
                         jnp.ones((_LANE, _LANE), jnp.float32)], axis=1)

    TB = _NSUB * _LANE
    G = R // _RB
    K = T // TB

    out = pl.pallas_call(
        _pcen_body,
        grid=(G, K),
        in_specs=[
            pl.BlockSpec((_RB, 8), lambda g, k: (g, 0)),
            pl.BlockSpec((_LANE, 2 * _LANE), lambda g, k: (0, 0)),
            pl.BlockSpec((_RB, TB), lambda g, k: (g, k)),
        ],
        out_specs=pl.BlockSpec((_RB, TB), lambda g, k: (g, k)),
        out_shape=jax.ShapeDtypeStruct((R, T), jnp.float32),
        scratch_shapes=[pltpu.VMEM((_RB, _LANE), jnp.float32)] * 8,
        compiler_params=pltpu.CompilerParams(
            dimension_semantics=("parallel", "arbitrary"),
        ),
    )(params, u, xr)

    return out.reshape(B, C, T)


# speedup vs baseline: 109.1545x; 1.0836x over previous
import jax
import jax.numpy as jnp
from jax.experimental import pallas as pl
from jax.experimental.pallas import tpu as pltpu

_EPS = 1e-06
_LANE = 128
_NSUB = 8
_RB = 2048


def _pcen_body(params_ref, u_ref, x_ref, o_ref,
               acb_ref, p_ref, winvp_ref, ap_ref,
               na_ref, invr_ref, delta_ref, dr_ref):
    k = pl.program_id(1)
    ones = jnp.zeros((_RB, _LANE), jnp.float32)

    @pl.when(k == 0)
    def _init():
        w = params_ref[:, 0:1]
        a = params_ref[:, 1:2]
        la = jnp.log(a)
        i = jax.lax.broadcasted_iota(jnp.int32, (1, _LANE), 1).astype(jnp.float32)
        p = jnp.exp(i * la)
        p_ref[...] = p
        winvp_ref[...] = w / p
        ap_ref[...] = jnp.exp(128.0 * la) + ones
        na_ref[...] = -params_ref[:, 2:3] + ones
        invr_ref[...] = params_ref[:, 3:4] + ones
        delta_ref[...] = params_ref[:, 4:5] + ones
        dr_ref[...] = params_ref[:, 5:6] + ones
        acb_ref[...] = a * x_ref[:, 0:1] + ones

    p = p_ref[...]
    winvp = winvp_ref[...]
    ap = ap_ref[...]
    na = na_ref[...]
    invr = invr_ref[...]
    delta = delta_ref[...]
    dr = dr_ref[...]
    u = u_ref[...]

    ac = acb_ref[...]
    for c in range(_NSUB):
        xs = x_ref[:, c * _LANE:(c + 1) * _LANE]
        s = xs * winvp
        both = jax.lax.dot_general(
            s, u, (((1,), (0,)), ((), ())),
            preferred_element_type=jnp.float32)
        cum = both[:, :_LANE]
        sb = both[:, _LANE:]
        ema = p * (cum + ac)
        den = jnp.exp2(na * jnp.log2(ema + _EPS))
        ac = ap * (sb + ac)
        q = xs * den + delta
        o_ref[:, c * _LANE:(c + 1) * _LANE] = jnp.exp2(invr * jnp.log2(q)) - dr
    acb_ref[...] = ac


@jax.jit
def kernel(x, alpha, delta, root, ema_weights):
    B, _, C, T = x.shape
    R = B * C
    xr = x.reshape(R, T)

    w = jnp.clip(ema_weights, 0.0, 0.2)
    a = 1.0 - w
    al = jnp.minimum(alpha, 1.0)
    r = jnp.maximum(root, 1.0)
    inv_r = 1.0 / r
    dr = delta ** inv_r
    pc = jnp.stack([w, a, al, inv_r, delta, dr,
                    jnp.zeros_like(w), jnp.zeros_like(w)], axis=1)
    params = jnp.tile(pc, (B, 1))

    u = jnp.concatenate([jnp.triu(jnp.ones((_LANE, _LANE), jnp.float32)),
                         jnp.ones((_LANE, _LANE), jnp.float32)], axis=1)

    TB = _NSUB * _LANE
    G = R // _RB
    K = T // TB

    out = pl.pallas_call(
        _pcen_body,
        grid=(G, K),
        in_specs=[
            pl.BlockSpec((_RB, 8), lambda g, k: (g, 0)),
            pl.BlockSpec((_LANE, 2 * _LANE), lambda g, k: (0, 0)),
            pl.BlockSpec((_RB, TB), lambda g, k: (g, k)),
        ],
        out_specs=pl.BlockSpec((_RB, TB), lambda g, k: (g, k)),
        out_shape=jax.ShapeDtypeStruct((R, T), jnp.float32),
        scratch_shapes=[pltpu.VMEM((_RB, _LANE), jnp.float32)] * 8,
        compiler_params=pltpu.CompilerParams(
            dimension_semantics=("parallel", "arbitrary"),
        ),
    )(params, u, xr)

    return out.reshape(B, C, T)
